# baseline (speedup 1.0000x reference)
import jax
import jax.numpy as jnp
from jax.experimental import pallas as pl
from jax.experimental.pallas import tpu as pltpu

_BN_EPS = 1e-5
_TB = 256
_VC = 256


def _round_up(x, m):
    return ((x + m - 1) // m) * m


def _nfm_kernel(featT_ref, fvT_ref,
                etT_ref, esqT_ref,
                w1T_ref, b1T_ref,
                w2T_ref, b2T_ref,
                wpT_ref, bp_ref,
                out_ref):
    k_dim, vocab_p = etT_ref.shape
    num_fields = featT_ref.shape[0]
    tb = featT_ref.shape[2]

    sT = jnp.zeros((k_dim, tb), jnp.float32)
    sqT = jnp.zeros((k_dim, tb), jnp.float32)

    for c in range(vocab_p // _VC):
        iota_c = jax.lax.broadcasted_iota(jnp.int32, (_VC, tb), 0) + c * _VC

        def body(f, carry):
            o1, o2 = carry
            featf = featT_ref[f]
            fvf = fvT_ref[f]
            m = jnp.where(iota_c == featf, fvf, 0.0)
            return o1 + m, o2 + m * m

        o1T, o2T = jax.lax.fori_loop(
            0, num_fields, body,
            (jnp.zeros((_VC, tb), jnp.float32),
             jnp.zeros((_VC, tb), jnp.float32)))

        et_c = etT_ref[:, c * _VC:(c + 1) * _VC]
        esq_c = esqT_ref[:, c * _VC:(c + 1) * _VC]
        sT = sT + jnp.dot(et_c, o1T, preferred_element_type=jnp.float32)
        sqT = sqT + jnp.dot(esq_c, o2T, preferred_element_type=jnp.float32)

    rawT = sT * sT - sqT

    h = jnp.dot(w1T_ref[...], rawT, preferred_element_type=jnp.float32)
    h = jnp.maximum(h + b1T_ref[...], 0.0)
    h = jnp.dot(w2T_ref[...], h, preferred_element_type=jnp.float32)
    h = jnp.maximum(h + b2T_ref[...], 0.0)
    out_ref[...] = (jnp.sum(h * wpT_ref[...], axis=0, keepdims=True)
                    + bp_ref[...])


def kernel(features, feature_values, embeddings, g0, b0, m0, v0,
           w1, bb1, g1, be1, m1, v1, w2, bb2, g2, be2, m2, v2, wp, bp):
    B, F = features.shape
    V, K = embeddings.shape
    H1 = w1.shape[1]
    H2 = w2.shape[1]

    inv0 = jax.lax.rsqrt(v0 + _BN_EPS)
    s0 = g0 * inv0
    t0 = b0 - m0 * s0

    inv1 = jax.lax.rsqrt(v1 + _BN_EPS)
    sc1 = g1 * inv1
    w1f = w1 * sc1
    b1f = (bb1 - m1) * sc1 + be1

    inv2 = jax.lax.rsqrt(v2 + _BN_EPS)
    sc2 = g2 * inv2
    w2f = w2 * sc2
    b2f = (bb2 - m2) * sc2 + be2

    wA = (0.5 * s0).reshape(K, 1) * w1f
    bA = t0 @ w1f + b1f

    Vp = _round_up(V, 128)
    table = embeddings.astype(jnp.float32)
    if Vp != V:
        table = jnp.pad(table, ((0, Vp - V), (0, 0)))
    etT = table.T
    esqT = (table * table).T

    Bp = _round_up(B, _TB)
    feat = features.astype(jnp.int32)
    fv = feature_values.astype(jnp.float32)
    if Bp != B:
        feat = jnp.pad(feat, ((0, Bp - B), (0, 0)))
        fv = jnp.pad(fv, ((0, Bp - B), (0, 0)))
    featT = feat.T.reshape(F, 1, Bp)
    fvT = fv.T.reshape(F, 1, Bp)

    weight_args = (etT, esqT,
                   wA.T, bA.reshape(H1, 1),
                   w2f.T, b2f.reshape(H2, 1),
                   wp.reshape(H2, 1), bp.reshape(1, 1))
    const2d = lambda i: (0, 0)
    weight_specs = [
        pl.BlockSpec((K, Vp), const2d), pl.BlockSpec((K, Vp), const2d),
        pl.BlockSpec((H1, K), const2d), pl.BlockSpec((H1, 1), const2d),
        pl.BlockSpec((H2, H1), const2d), pl.BlockSpec((H2, 1), const2d),
        pl.BlockSpec((H2, 1), const2d), pl.BlockSpec((1, 1), const2d),
    ]

    flops = Bp * (4 * Vp * K + 2 * K * H1 + 2 * H1 * H2 + 2 * H2 + 6 * K)
    bytes_accessed = 4 * (2 * Bp * F + Bp + 2 * Vp * K
                          + K * H1 + H1 * H2 + H1 + 2 * H2 + 1)

    out = pl.pallas_call(
        _nfm_kernel,
        out_shape=jax.ShapeDtypeStruct((1, Bp), jnp.float32),
        grid=(Bp // _TB,),
        in_specs=[
            pl.BlockSpec((F, 1, _TB), lambda i: (0, 0, i)),
            pl.BlockSpec((F, 1, _TB), lambda i: (0, 0, i)),
        ] + weight_specs,
        out_specs=pl.BlockSpec((1, _TB), lambda i: (0, i)),
        compiler_params=pltpu.CompilerParams(
            dimension_semantics=("parallel",),
            vmem_limit_bytes=64 * 1024 * 1024,
        ),
        cost_estimate=pl.CostEstimate(
            flops=int(flops), transcendentals=0,
            bytes_accessed=int(bytes_accessed)),
    )(featT, fvT, *weight_args)
    return out[0, :B]


# speedup vs baseline: 1.7003x; 1.7003x over previous
import jax
import jax.numpy as jnp
from jax.experimental import pallas as pl
from jax.experimental.pallas import tpu as pltpu

_BN_EPS = 1e-5
_TB = 256
_VC = 128
_FG = 8


def _round_up(x, m):
    return ((x + m - 1) // m) * m


def _nfm_kernel(featT_ref, fvT_ref,
                et3_ref, esq3_ref,
                w1T_ref, b1T_ref,
                w2T_ref, b2T_ref,
                wpT_ref, bp_ref,
                out_ref):
    n_chunks, _, k_dim = et3_ref.shape
    num_fields = featT_ref.shape[0]
    tb = featT_ref.shape[2]
    iota0 = jax.lax.broadcasted_iota(jnp.int32, (_VC, tb), 0)
    dim_nums = (((0,), (0,)), ((), ()))

    def chunk_body(c, carry):
        sT, sqT = carry
        iota_c = iota0 + c * _VC

        def field_body(g, inner):
            o1, o2 = inner
            for j in range(_FG):
                f = g * _FG + j
                m = jnp.where(iota_c == featT_ref[f], fvT_ref[f], 0.0)
                o1 = o1 + m
                o2 = o2 + m * m
            return o1, o2

        o1T, o2T = jax.lax.fori_loop(
            0, num_fields // _FG, field_body,
            (jnp.zeros((_VC, tb), jnp.float32),
             jnp.zeros((_VC, tb), jnp.float32)))

        sT = sT + jax.lax.dot_general(
            et3_ref[c], o1T, dim_nums, preferred_element_type=jnp.float32)
        sqT = sqT + jax.lax.dot_general(
            esq3_ref[c], o2T, dim_nums, preferred_element_type=jnp.float32)
        return sT, sqT

    sT, sqT = jax.lax.fori_loop(
        0, n_chunks, chunk_body,
        (jnp.zeros((k_dim, tb), jnp.float32),
         jnp.zeros((k_dim, tb), jnp.float32)))

    rawT = sT * sT - sqT

    h = jnp.dot(w1T_ref[...], rawT, preferred_element_type=jnp.float32)
    h = jnp.maximum(h + b1T_ref[...], 0.0)
    h = jnp.dot(w2T_ref[...], h, preferred_element_type=jnp.float32)
    h = jnp.maximum(h + b2T_ref[...], 0.0)
    out_ref[...] = (jnp.sum(h * wpT_ref[...], axis=0, keepdims=True)
                    + bp_ref[...])


def kernel(features, feature_values, embeddings, g0, b0, m0, v0,
           w1, bb1, g1, be1, m1, v1, w2, bb2, g2, be2, m2, v2, wp, bp):
    B, F = features.shape
    V, K = embeddings.shape
    H1 = w1.shape[1]
    H2 = w2.shape[1]

    inv0 = jax.lax.rsqrt(v0 + _BN_EPS)
    s0 = g0 * inv0
    t0 = b0 - m0 * s0

    inv1 = jax.lax.rsqrt(v1 + _BN_EPS)
    sc1 = g1 * inv1
    w1f = w1 * sc1
    b1f = (bb1 - m1) * sc1 + be1

    inv2 = jax.lax.rsqrt(v2 + _BN_EPS)
    sc2 = g2 * inv2
    w2f = w2 * sc2
    b2f = (bb2 - m2) * sc2 + be2

    wA = (0.5 * s0).reshape(K, 1) * w1f
    bA = t0 @ w1f + b1f

    Vp = _round_up(V, _VC)
    table = embeddings.astype(jnp.float32)
    if Vp != V:
        table = jnp.pad(table, ((0, Vp - V), (0, 0)))
    et3 = table.reshape(Vp // _VC, _VC, K)
    esq3 = (table * table).reshape(Vp // _VC, _VC, K)

    Bp = _round_up(B, _TB)
    Fp = _round_up(F, _FG)
    feat = features.astype(jnp.int32)
    fv = feature_values.astype(jnp.float32)
    if Bp != B or Fp != F:
        feat = jnp.pad(feat, ((0, Bp - B), (0, Fp - F)))
        fv = jnp.pad(fv, ((0, Bp - B), (0, Fp - F)))
    featT = feat.T.reshape(Fp, 1, Bp)
    fvT = fv.T.reshape(Fp, 1, Bp)

    weight_args = (et3, esq3,
                   wA.T, bA.reshape(H1, 1),
                   w2f.T, b2f.reshape(H2, 1),
                   wp.reshape(H2, 1), bp.reshape(1, 1))
    const2d = lambda i: (0, 0)
    const3d = lambda i: (0, 0, 0)
    weight_specs = [
        pl.BlockSpec((Vp // _VC, _VC, K), const3d),
        pl.BlockSpec((Vp // _VC, _VC, K), const3d),
        pl.BlockSpec((H1, K), const2d), pl.BlockSpec((H1, 1), const2d),
        pl.BlockSpec((H2, H1), const2d), pl.BlockSpec((H2, 1), const2d),
        pl.BlockSpec((H2, 1), const2d), pl.BlockSpec((1, 1), const2d),
    ]

    flops = Bp * (4 * Vp * K + 2 * K * H1 + 2 * H1 * H2 + 2 * H2 + 6 * K)
    bytes_accessed = 4 * (2 * Bp * F + Bp + 2 * Vp * K
                          + K * H1 + H1 * H2 + H1 + 2 * H2 + 1)

    out = pl.pallas_call(
        _nfm_kernel,
        out_shape=jax.ShapeDtypeStruct((1, Bp), jnp.float32),
        grid=(Bp // _TB,),
        in_specs=[
            pl.BlockSpec((Fp, 1, _TB), lambda i: (0, 0, i)),
            pl.BlockSpec((Fp, 1, _TB), lambda i: (0, 0, i)),
        ] + weight_specs,
        out_specs=pl.BlockSpec((1, _TB), lambda i: (0, i)),
        compiler_params=pltpu.CompilerParams(
            dimension_semantics=("parallel",),
            vmem_limit_bytes=64 * 1024 * 1024,
        ),
        cost_estimate=pl.CostEstimate(
            flops=int(flops), transcendentals=0,
            bytes_accessed=int(bytes_accessed)),
    )(featT, fvT, *weight_args)
    return out[0, :B]


# speedup vs baseline: 2.3731x; 1.3957x over previous
import jax
import jax.numpy as jnp
from jax.experimental import pallas as pl
from jax.experimental.pallas import tpu as pltpu

_BN_EPS = 1e-5
_TB = 256
_VC = 128
_FG = 8


def _round_up(x, m):
    return ((x + m - 1) // m) * m


def _nfm_kernel(featT_ref, fvT_ref,
                et3_ref, esq3_ref,
                w1T_ref, b1T_ref,
                w2T_ref, b2T_ref,
                wpT_ref, bp_ref,
                out_ref):
    n_chunks, _, k_dim = et3_ref.shape
    num_fields = featT_ref.shape[0]
    tb = featT_ref.shape[2]
    iota0 = jax.lax.broadcasted_iota(jnp.int32, (_VC, tb), 0)
    dim_nums = (((0,), (0,)), ((), ()))

    def chunk_body(c, carry):
        sT, sqT = carry
        iota_c = iota0 + c * _VC

        o1T = jnp.zeros((_VC, tb), jnp.float32)
        o2T = jnp.zeros((_VC, tb), jnp.float32)
        for f in range(num_fields):
            m = jnp.where(iota_c == featT_ref[f], fvT_ref[f], 0.0)
            o1T = o1T + m
            o2T = o2T + m * m

        sT = sT + jax.lax.dot_general(
            et3_ref[c], o1T, dim_nums, preferred_element_type=jnp.float32)
        sqT = sqT + jax.lax.dot_general(
            esq3_ref[c], o2T, dim_nums, preferred_element_type=jnp.float32)
        return sT, sqT

    sT, sqT = jax.lax.fori_loop(
        0, n_chunks, chunk_body,
        (jnp.zeros((k_dim, tb), jnp.float32),
         jnp.zeros((k_dim, tb), jnp.float32)))

    rawT = sT * sT - sqT

    h = jnp.dot(w1T_ref[...], rawT, preferred_element_type=jnp.float32)
    h = jnp.maximum(h + b1T_ref[...], 0.0)
    h = jnp.dot(w2T_ref[...], h, preferred_element_type=jnp.float32)
    h = jnp.maximum(h + b2T_ref[...], 0.0)
    out_ref[...] = (jnp.sum(h * wpT_ref[...], axis=0, keepdims=True)
                    + bp_ref[...])


def kernel(features, feature_values, embeddings, g0, b0, m0, v0,
           w1, bb1, g1, be1, m1, v1, w2, bb2, g2, be2, m2, v2, wp, bp):
    B, F = features.shape
    V, K = embeddings.shape
    H1 = w1.shape[1]
    H2 = w2.shape[1]

    inv0 = jax.lax.rsqrt(v0 + _BN_EPS)
    s0 = g0 * inv0
    t0 = b0 - m0 * s0

    inv1 = jax.lax.rsqrt(v1 + _BN_EPS)
    sc1 = g1 * inv1
    w1f = w1 * sc1
    b1f = (bb1 - m1) * sc1 + be1

    inv2 = jax.lax.rsqrt(v2 + _BN_EPS)
    sc2 = g2 * inv2
    w2f = w2 * sc2
    b2f = (bb2 - m2) * sc2 + be2

    wA = (0.5 * s0).reshape(K, 1) * w1f
    bA = t0 @ w1f + b1f

    Vp = _round_up(V, _VC)
    table = embeddings.astype(jnp.float32)
    if Vp != V:
        table = jnp.pad(table, ((0, Vp - V), (0, 0)))
    et3 = table.reshape(Vp // _VC, _VC, K)
    esq3 = (table * table).reshape(Vp // _VC, _VC, K)

    Bp = _round_up(B, _TB)
    Fp = _round_up(F, _FG)
    feat = features.astype(jnp.int32)
    fv = feature_values.astype(jnp.float32)
    if Bp != B or Fp != F:
        feat = jnp.pad(feat, ((0, Bp - B), (0, Fp - F)))
        fv = jnp.pad(fv, ((0, Bp - B), (0, Fp - F)))
    featT = feat.T.reshape(Fp, 1, Bp)
    fvT = fv.T.reshape(Fp, 1, Bp)

    weight_args = (et3, esq3,
                   wA.T, bA.reshape(H1, 1),
                   w2f.T, b2f.reshape(H2, 1),
                   wp.reshape(H2, 1), bp.reshape(1, 1))
    const2d = lambda i: (0, 0)
    const3d = lambda i: (0, 0, 0)
    weight_specs = [
        pl.BlockSpec((Vp // _VC, _VC, K), const3d),
        pl.BlockSpec((Vp // _VC, _VC, K), const3d),
        pl.BlockSpec((H1, K), const2d), pl.BlockSpec((H1, 1), const2d),
        pl.BlockSpec((H2, H1), const2d), pl.BlockSpec((H2, 1), const2d),
        pl.BlockSpec((H2, 1), const2d), pl.BlockSpec((1, 1), const2d),
    ]

    flops = Bp * (4 * Vp * K + 2 * K * H1 + 2 * H1 * H2 + 2 * H2 + 6 * K)
    bytes_accessed = 4 * (2 * Bp * F + Bp + 2 * Vp * K
                          + K * H1 + H1 * H2 + H1 + 2 * H2 + 1)

    out = pl.pallas_call(
        _nfm_kernel,
        out_shape=jax.ShapeDtypeStruct((1, Bp), jnp.float32),
        grid=(Bp // _TB,),
        in_specs=[
            pl.BlockSpec((Fp, 1, _TB), lambda i: (0, 0, i)),
            pl.BlockSpec((Fp, 1, _TB), lambda i: (0, 0, i)),
        ] + weight_specs,
        out_specs=pl.BlockSpec((1, _TB), lambda i: (0, i)),
        compiler_params=pltpu.CompilerParams(
            dimension_semantics=("parallel",),
            vmem_limit_bytes=64 * 1024 * 1024,
        ),
        cost_estimate=pl.CostEstimate(
            flops=int(flops), transcendentals=0,
            bytes_accessed=int(bytes_accessed)),
    )(featT, fvT, *weight_args)
    return out[0, :B]
